# baseline (speedup 1.0000x reference)
import functools

import jax
import jax.numpy as jnp
import numpy as np
from jax import lax
from jax.experimental import pallas as pl
from jax.experimental.pallas import tpu as pltpu
from jax.experimental.pallas import tpu_sc as plsc

_NC = 2
_NS = 16
_NW = _NC * _NS


def _sc_gather(idxr, emb, TB, D):
    NBLK = idxr.shape[1]
    per_w = NBLK * 128

    mesh = plsc.VectorSubcoreMesh(core_axis_name="c", subcore_axis_name="s")

    @functools.partial(
        pl.kernel,
        out_type=jax.ShapeDtypeStruct((TB, D), jnp.float32),
        mesh=mesh,
        scratch_types=[
            pltpu.VMEM((NBLK, 128), jnp.int32),
            pltpu.VMEM((128, D), jnp.float32),
            pltpu.SemaphoreType.DMA,
        ],
    )
    def gather_k(idx_hbm, emb_hbm, out_hbm, idx_v, rows_v, sem):
        wid = lax.axis_index("s") * _NC + lax.axis_index("c")
        base = wid * per_w
        pltpu.sync_copy(idx_hbm.at[wid], idx_v)

        def body(j, carry):
            pltpu.async_copy(emb_hbm.at[idx_v.at[j]], rows_v, sem).wait()
            pltpu.sync_copy(rows_v, out_hbm.at[pl.ds(base + j * 128, 128)])
            return carry

        lax.fori_loop(0, NBLK, body, 0)

    return gather_k(idxr, emb)


def _pack_weights(Wf, Wb, Uf, Ub, bf, bb, U):
    def inter(Mf, Mb):
        R = Mf.shape[0]
        return jnp.concatenate(
            [Mf.reshape(R, 4, U), Mb.reshape(R, 4, U)], axis=2
        ).reshape(R, 8 * U)

    WU = jnp.concatenate([
        inter(Wf, jnp.zeros_like(Wf)),
        inter(jnp.zeros_like(Wb), Wb),
        inter(Uf, jnp.zeros_like(Uf)),
        inter(jnp.zeros_like(Ub), Ub),
    ], axis=0)
    bc = jnp.concatenate(
        [bf.reshape(4, U), bb.reshape(4, U)], axis=1).reshape(1, 8 * U)
    return WU, bc


def _gates(Z, C, U):
    H2 = 2 * U
    I = jax.nn.sigmoid(Z[:, 0:H2])
    F = jax.nn.sigmoid(Z[:, H2:2 * H2])
    G = jnp.tanh(Z[:, 2 * H2:3 * H2])
    O = jax.nn.sigmoid(Z[:, 3 * H2:4 * H2])
    Cn = F * C + I * G
    Hn = O * jnp.tanh(Cn)
    return Hn, Cn


def _layer1(xe, WU1, bc1, B, T, D, U):
    H2 = 2 * U

    def body(xf_ref, xb_ref, wu_ref, b_ref, hf_ref, hb_ref, Hs, Cs):
        s = pl.program_id(0)

        @pl.when(s == 0)
        def _():
            Hs[...] = jnp.zeros_like(Hs)
            Cs[...] = jnp.zeros_like(Cs)

        X = jnp.concatenate([xf_ref[0], xb_ref[0], Hs[...]], axis=1)
        Z = jnp.dot(X, wu_ref[...], preferred_element_type=jnp.float32, precision=lax.Precision.HIGHEST)
        Z = Z + b_ref[...]
        Hn, Cn = _gates(Z, Cs[...], U)
        Cs[...] = Cn
        Hs[...] = Hn
        hf_ref[0] = Hn[:, 0:U]
        hb_ref[0] = Hn[:, U:H2]

    return pl.pallas_call(
        body,
        grid=(T,),
        in_specs=[
            pl.BlockSpec((1, B, D), lambda s: (s, 0, 0)),
            pl.BlockSpec((1, B, D), lambda s: (T - 1 - s, 0, 0)),
            pl.BlockSpec(WU1.shape, lambda s: (0, 0)),
            pl.BlockSpec((1, 4 * H2), lambda s: (0, 0)),
        ],
        out_specs=[
            pl.BlockSpec((1, B, U), lambda s: (s, 0, 0)),
            pl.BlockSpec((1, B, U), lambda s: (T - 1 - s, 0, 0)),
        ],
        out_shape=[
            jax.ShapeDtypeStruct((T, B, U), jnp.float32),
            jax.ShapeDtypeStruct((T, B, U), jnp.float32),
        ],
        scratch_shapes=[
            pltpu.VMEM((B, H2), jnp.float32),
            pltpu.VMEM((B, H2), jnp.float32),
        ],
        compiler_params=pltpu.CompilerParams(
            dimension_semantics=("arbitrary",)),
    )(xe, xe, WU1, bc1)


def _layer2_dense(h1f, h1b, WU2, bc2, Wdf, Wdb, bdp, B, T, U):
    H2 = 2 * U

    def body(hfs_ref, hbs_ref, hfr_ref, hbr_ref, wu_ref, b_ref,
             wdf_ref, wdb_ref, bd_ref, out_ref, Hs, Cs, Acc):
        s = pl.program_id(0)

        @pl.when(s == 0)
        def _():
            Hs[...] = jnp.zeros_like(Hs)
            Cs[...] = jnp.zeros_like(Cs)
            Acc[...] = jnp.zeros_like(Acc)

        x2f = jnp.concatenate([hfs_ref[0], hbs_ref[0]], axis=1)
        x2b = jnp.concatenate([hfr_ref[0], hbr_ref[0]], axis=1)
        X = jnp.concatenate([x2f, x2b, Hs[...]], axis=1)
        Z = jnp.dot(X, wu_ref[...], preferred_element_type=jnp.float32, precision=lax.Precision.HIGHEST)
        Z = Z + b_ref[...]
        Hn, Cn = _gates(Z, Cs[...], U)
        Cs[...] = Cn
        Hs[...] = Hn
        Wds = jnp.concatenate([wdf_ref[0], wdb_ref[0]], axis=0)
        Acc[...] += jnp.dot(Hn, Wds, preferred_element_type=jnp.float32, precision=lax.Precision.HIGHEST)

        @pl.when(s == T - 1)
        def _():
            z = Acc[...] + bd_ref[...]
            lane = lax.broadcasted_iota(jnp.int32, z.shape, 1)
            valid = lane < 3
            zm = jnp.where(valid, z, -jnp.inf)
            m = jnp.max(zm, axis=1, keepdims=True)
            e = jnp.where(valid, jnp.exp(zm - m), 0.0)
            out_ref[...] = e / jnp.sum(e, axis=1, keepdims=True)

    return pl.pallas_call(
        body,
        grid=(T,),
        in_specs=[
            pl.BlockSpec((1, B, U), lambda s: (s, 0, 0)),
            pl.BlockSpec((1, B, U), lambda s: (s, 0, 0)),
            pl.BlockSpec((1, B, U), lambda s: (T - 1 - s, 0, 0)),
            pl.BlockSpec((1, B, U), lambda s: (T - 1 - s, 0, 0)),
            pl.BlockSpec(WU2.shape, lambda s: (0, 0)),
            pl.BlockSpec((1, 4 * H2), lambda s: (0, 0)),
            pl.BlockSpec((1, U, 128), lambda s: (s, 0, 0)),
            pl.BlockSpec((1, U, 128), lambda s: (T - 1 - s, 0, 0)),
            pl.BlockSpec((1, 128), lambda s: (0, 0)),
        ],
        out_specs=pl.BlockSpec((B, 128), lambda s: (0, 0)),
        out_shape=jax.ShapeDtypeStruct((B, 128), jnp.float32),
        scratch_shapes=[
            pltpu.VMEM((B, H2), jnp.float32),
            pltpu.VMEM((B, H2), jnp.float32),
            pltpu.VMEM((B, 128), jnp.float32),
        ],
        compiler_params=pltpu.CompilerParams(
            dimension_semantics=("arbitrary",)),
    )(h1f, h1b, h1f, h1b, WU2, bc2, Wdf, Wdb, bdp)


def kernel(x, emb, W1f, U1f, b1f, W1b, U1b, b1b,
           W2f, U2f, b2f, W2b, U2b, b2b, Wd, bd):
    B, T = x.shape
    V, D = emb.shape
    U = U1f.shape[0]
    TB = T * B
    NBLK = TB // (_NW * 128)

    idxr = x.T.reshape(_NW, NBLK, 128)
    xe = _sc_gather(idxr, emb, TB, D).reshape(T, B, D)

    WU1, bc1 = _pack_weights(W1f, W1b, U1f, U1b, b1f, b1b, U)
    WU2, bc2 = _pack_weights(W2f, W2b, U2f, U2b, b2f, b2b, U)
    Wd3 = Wd.reshape(T, 2 * U, 3)
    Wdf = jnp.zeros((T, U, 128), jnp.float32).at[:, :, 0:3].set(Wd3[:, 0:U, :])
    Wdb = jnp.zeros((T, U, 128), jnp.float32).at[:, :, 0:3].set(Wd3[:, U:, :])
    bdp = jnp.zeros((1, 128), jnp.float32).at[0, 0:3].set(bd)

    h1f, h1b = _layer1(xe, WU1, bc1, B, T, D, U)
    probs_pad = _layer2_dense(h1f, h1b, WU2, bc2, Wdf, Wdb, bdp, B, T, U)
    return probs_pad[:, 0:3]


# speedup vs baseline: 2.7209x; 2.7209x over previous
import functools

import jax
import jax.numpy as jnp
import numpy as np
from jax import lax
from jax.experimental import pallas as pl
from jax.experimental.pallas import tpu as pltpu
from jax.experimental.pallas import tpu_sc as plsc

_NC = 2
_NS = 16
_NW = _NC * _NS


def _sc_gather(idxr, emb, TB, D):
    NBLK = idxr.shape[1]
    per_w = NBLK * 128

    mesh = plsc.VectorSubcoreMesh(core_axis_name="c", subcore_axis_name="s")

    @functools.partial(
        pl.kernel,
        out_type=jax.ShapeDtypeStruct((TB, D), jnp.float32),
        mesh=mesh,
        scratch_types=[
            pltpu.VMEM((NBLK, 128), jnp.int32),
            pltpu.VMEM((128, D), jnp.float32),
            pltpu.SemaphoreType.DMA,
        ],
    )
    def gather_k(idx_hbm, emb_hbm, out_hbm, idx_v, rows_v, sem):
        wid = lax.axis_index("s") * _NC + lax.axis_index("c")
        base = wid * per_w
        pltpu.sync_copy(idx_hbm.at[wid], idx_v)

        def body(j, carry):
            pltpu.async_copy(emb_hbm.at[idx_v.at[j]], rows_v, sem).wait()
            pltpu.sync_copy(rows_v, out_hbm.at[pl.ds(base + j * 128, 128)])
            return carry

        lax.fori_loop(0, NBLK, body, 0)

    return gather_k(idxr, emb)


def _pack_weights(Wf, Wb, Uf, Ub, bf, bb, U):
    def inter(Mf, Mb):
        R = Mf.shape[0]
        return jnp.concatenate(
            [Mf.reshape(R, 4, U), Mb.reshape(R, 4, U)], axis=2
        ).reshape(R, 8 * U)

    Wcat = jnp.concatenate([
        inter(Wf, jnp.zeros_like(Wf)),
        inter(jnp.zeros_like(Wb), Wb),
    ], axis=0)
    Ucat = jnp.concatenate([
        inter(Uf, jnp.zeros_like(Uf)),
        inter(jnp.zeros_like(Ub), Ub),
    ], axis=0)
    bc = jnp.concatenate(
        [bf.reshape(4, U), bb.reshape(4, U)], axis=1).reshape(1, 8 * U)
    return Wcat, Ucat, bc


def _gates(Z, C, U):
    H2 = 2 * U
    I = jax.nn.sigmoid(Z[:, 0:H2])
    F = jax.nn.sigmoid(Z[:, H2:2 * H2])
    G = jnp.tanh(Z[:, 2 * H2:3 * H2])
    O = jax.nn.sigmoid(Z[:, 3 * H2:4 * H2])
    Cn = F * C + I * G
    Hn = O * jnp.tanh(Cn)
    return Hn, Cn


def _layer1(xe, W1, U1, bc1, B, T, D, U):
    H2 = 2 * U

    def body(xf_ref, xb_ref, w_ref, u_ref, b_ref, hf_ref, hb_ref, Hs, Cs):
        s = pl.program_id(0)

        @pl.when(s == 0)
        def _():
            Hs[...] = jnp.zeros_like(Hs)
            Cs[...] = jnp.zeros_like(Cs)

        X = jnp.concatenate([xf_ref[0], xb_ref[0]], axis=1)
        Z = (jnp.dot(X, w_ref[...], preferred_element_type=jnp.float32)
             + jnp.dot(Hs[...], u_ref[...], preferred_element_type=jnp.float32)
             + b_ref[...])
        Hn, Cn = _gates(Z, Cs[...], U)
        Cs[...] = Cn
        Hs[...] = Hn
        hf_ref[0] = Hn[:, 0:U]
        hb_ref[0] = Hn[:, U:H2]

    return pl.pallas_call(
        body,
        grid=(T,),
        in_specs=[
            pl.BlockSpec((1, B, D), lambda s: (s, 0, 0)),
            pl.BlockSpec((1, B, D), lambda s: (T - 1 - s, 0, 0)),
            pl.BlockSpec(W1.shape, lambda s: (0, 0)),
            pl.BlockSpec(U1.shape, lambda s: (0, 0)),
            pl.BlockSpec((1, 4 * H2), lambda s: (0, 0)),
        ],
        out_specs=[
            pl.BlockSpec((1, B, U), lambda s: (s, 0, 0)),
            pl.BlockSpec((1, B, U), lambda s: (T - 1 - s, 0, 0)),
        ],
        out_shape=[
            jax.ShapeDtypeStruct((T, B, U), jnp.float32),
            jax.ShapeDtypeStruct((T, B, U), jnp.float32),
        ],
        scratch_shapes=[
            pltpu.VMEM((B, H2), jnp.float32),
            pltpu.VMEM((B, H2), jnp.float32),
        ],
        compiler_params=pltpu.CompilerParams(
            dimension_semantics=("arbitrary",)),
    )(xe, xe, W1, U1, bc1)


def _layer2_dense(h1f, h1b, W2, U2, bc2, Wdf, Wdb, bdp, B, T, U):
    H2 = 2 * U

    def body(hfs_ref, hbs_ref, hfr_ref, hbr_ref, w_ref, u_ref, b_ref,
             wdf_ref, wdb_ref, bd_ref, out_ref, Hs, Cs, Acc):
        s = pl.program_id(0)

        @pl.when(s == 0)
        def _():
            Hs[...] = jnp.zeros_like(Hs)
            Cs[...] = jnp.zeros_like(Cs)
            Acc[...] = jnp.zeros_like(Acc)

        X = jnp.concatenate(
            [hfs_ref[0], hbs_ref[0], hfr_ref[0], hbr_ref[0]], axis=1)
        Z = (jnp.dot(X, w_ref[...], preferred_element_type=jnp.float32)
             + jnp.dot(Hs[...], u_ref[...], preferred_element_type=jnp.float32)
             + b_ref[...])
        Hn, Cn = _gates(Z, Cs[...], U)
        Cs[...] = Cn
        Hs[...] = Hn
        Wds = jnp.concatenate([wdf_ref[0], wdb_ref[0]], axis=0)
        Acc[...] += jnp.dot(Hn, Wds, preferred_element_type=jnp.float32)

        @pl.when(s == T - 1)
        def _():
            z = Acc[...] + bd_ref[...]
            lane = lax.broadcasted_iota(jnp.int32, z.shape, 1)
            valid = lane < 3
            zm = jnp.where(valid, z, -jnp.inf)
            m = jnp.max(zm, axis=1, keepdims=True)
            e = jnp.where(valid, jnp.exp(zm - m), 0.0)
            out_ref[...] = e / jnp.sum(e, axis=1, keepdims=True)

    return pl.pallas_call(
        body,
        grid=(T,),
        in_specs=[
            pl.BlockSpec((1, B, U), lambda s: (s, 0, 0)),
            pl.BlockSpec((1, B, U), lambda s: (s, 0, 0)),
            pl.BlockSpec((1, B, U), lambda s: (T - 1 - s, 0, 0)),
            pl.BlockSpec((1, B, U), lambda s: (T - 1 - s, 0, 0)),
            pl.BlockSpec(W2.shape, lambda s: (0, 0)),
            pl.BlockSpec(U2.shape, lambda s: (0, 0)),
            pl.BlockSpec((1, 4 * H2), lambda s: (0, 0)),
            pl.BlockSpec((1, U, 128), lambda s: (s, 0, 0)),
            pl.BlockSpec((1, U, 128), lambda s: (T - 1 - s, 0, 0)),
            pl.BlockSpec((1, 128), lambda s: (0, 0)),
        ],
        out_specs=pl.BlockSpec((B, 128), lambda s: (0, 0)),
        out_shape=jax.ShapeDtypeStruct((B, 128), jnp.float32),
        scratch_shapes=[
            pltpu.VMEM((B, H2), jnp.float32),
            pltpu.VMEM((B, H2), jnp.float32),
            pltpu.VMEM((B, 128), jnp.float32),
        ],
        compiler_params=pltpu.CompilerParams(
            dimension_semantics=("arbitrary",)),
    )(h1f, h1b, h1f, h1b, W2, U2, bc2, Wdf, Wdb, bdp)


def kernel(x, emb, W1f, U1f, b1f, W1b, U1b, b1b,
           W2f, U2f, b2f, W2b, U2b, b2b, Wd, bd):
    B, T = x.shape
    V, D = emb.shape
    U = U1f.shape[0]
    TB = T * B
    NBLK = TB // (_NW * 128)

    idxr = x.T.reshape(_NW, NBLK, 128)
    xe = _sc_gather(idxr, emb, TB, D).reshape(T, B, D)

    Wc1, Uc1, bc1 = _pack_weights(W1f, W1b, U1f, U1b, b1f, b1b, U)
    Wc2, Uc2, bc2 = _pack_weights(W2f, W2b, U2f, U2b, b2f, b2b, U)
    Wd3 = Wd.reshape(T, 2 * U, 3)
    Wdf = jnp.zeros((T, U, 128), jnp.float32).at[:, :, 0:3].set(Wd3[:, 0:U, :])
    Wdb = jnp.zeros((T, U, 128), jnp.float32).at[:, :, 0:3].set(Wd3[:, U:, :])
    bdp = jnp.zeros((1, 128), jnp.float32).at[0, 0:3].set(bd)

    h1f, h1b = _layer1(xe, Wc1, Uc1, bc1, B, T, D, U)
    probs_pad = _layer2_dense(h1f, h1b, Wc2, Uc2, bc2, Wdf, Wdb, bdp, B, T, U)
    return probs_pad[:, 0:3]


# speedup vs baseline: 2.9965x; 1.1013x over previous
import functools

import jax
import jax.numpy as jnp
import numpy as np
from jax import lax
from jax.experimental import pallas as pl
from jax.experimental.pallas import tpu as pltpu
from jax.experimental.pallas import tpu_sc as plsc

_NC = 2
_NS = 16
_NW = _NC * _NS


def _sc_gather(idxr, emb, TB, D):
    NBLK = idxr.shape[1]
    per_w = NBLK * 128

    mesh = plsc.VectorSubcoreMesh(core_axis_name="c", subcore_axis_name="s")

    @functools.partial(
        pl.kernel,
        out_type=jax.ShapeDtypeStruct((TB, D), jnp.float32),
        mesh=mesh,
        scratch_types=[
            pltpu.VMEM((NBLK, 128), jnp.int32),
            pltpu.VMEM((128, D), jnp.float32),
            pltpu.SemaphoreType.DMA,
        ],
    )
    def gather_k(idx_hbm, emb_hbm, out_hbm, idx_v, rows_v, sem):
        wid = lax.axis_index("s") * _NC + lax.axis_index("c")
        base = wid * per_w
        pltpu.sync_copy(idx_hbm.at[wid], idx_v)

        def body(j, carry):
            pltpu.async_copy(emb_hbm.at[idx_v.at[j]], rows_v, sem).wait()
            pltpu.sync_copy(rows_v, out_hbm.at[pl.ds(base + j * 128, 128)])
            return carry

        lax.fori_loop(0, NBLK, body, 0)

    return gather_k(idxr, emb)


def _pack_weights(Wf, Wb, Uf, Ub, bf, bb, U):
    def inter(Mf, Mb):
        R = Mf.shape[0]
        return jnp.concatenate(
            [Mf.reshape(R, 4, U), Mb.reshape(R, 4, U)], axis=2
        ).reshape(R, 8 * U)

    Wcat = jnp.concatenate([
        inter(Wf, jnp.zeros_like(Wf)),
        inter(jnp.zeros_like(Wb), Wb),
    ], axis=0)
    Ucat = jnp.concatenate([
        inter(Uf, jnp.zeros_like(Uf)),
        inter(jnp.zeros_like(Ub), Ub),
    ], axis=0)
    bc = jnp.concatenate(
        [bf.reshape(4, U), bb.reshape(4, U)], axis=1).reshape(1, 8 * U)
    return Wcat, Ucat, bc


def _gates(Z, C, U):
    H2 = 2 * U
    I = jax.nn.sigmoid(Z[:, 0:H2])
    F = jax.nn.sigmoid(Z[:, H2:2 * H2])
    G = jnp.tanh(Z[:, 2 * H2:3 * H2])
    O = jax.nn.sigmoid(Z[:, 3 * H2:4 * H2])
    Cn = F * C + I * G
    Hn = O * jnp.tanh(Cn)
    return Hn, Cn


def _layer1(xe, W1, U1, bc1, B, T, D, U):
    H2 = 2 * U

    def body(xf_ref, xb_ref, w_ref, u_ref, b_ref, hf_ref, hb_ref, Hs, Cs):
        s = pl.program_id(0)

        @pl.when(s == 0)
        def _():
            Hs[...] = jnp.zeros_like(Hs)
            Cs[...] = jnp.zeros_like(Cs)

        X = jnp.concatenate([xf_ref[0], xb_ref[0]], axis=1)
        Z = (jnp.dot(X, w_ref[...], preferred_element_type=jnp.float32)
             + jnp.dot(Hs[...], u_ref[...], preferred_element_type=jnp.float32)
             + b_ref[...])
        Hn, Cn = _gates(Z, Cs[...], U)
        Cs[...] = Cn
        Hs[...] = Hn
        Hb = Hn.astype(jnp.bfloat16)
        hf_ref[0] = Hb[:, 0:U]
        hb_ref[0] = Hb[:, U:H2]

    return pl.pallas_call(
        body,
        grid=(T,),
        in_specs=[
            pl.BlockSpec((1, B, D), lambda s: (s, 0, 0)),
            pl.BlockSpec((1, B, D), lambda s: (T - 1 - s, 0, 0)),
            pl.BlockSpec(W1.shape, lambda s: (0, 0)),
            pl.BlockSpec(U1.shape, lambda s: (0, 0)),
            pl.BlockSpec((1, 4 * H2), lambda s: (0, 0)),
        ],
        out_specs=[
            pl.BlockSpec((1, B, U), lambda s: (s, 0, 0)),
            pl.BlockSpec((1, B, U), lambda s: (T - 1 - s, 0, 0)),
        ],
        out_shape=[
            jax.ShapeDtypeStruct((T, B, U), jnp.bfloat16),
            jax.ShapeDtypeStruct((T, B, U), jnp.bfloat16),
        ],
        scratch_shapes=[
            pltpu.VMEM((B, H2), jnp.float32),
            pltpu.VMEM((B, H2), jnp.float32),
        ],
        compiler_params=pltpu.CompilerParams(
            dimension_semantics=("arbitrary",)),
    )(xe, xe, W1, U1, bc1)


def _layer2_dense(h1f, h1b, W2, U2, bc2, Wdf, Wdb, bdp, B, T, U):
    H2 = 2 * U

    def body(hfs_ref, hbs_ref, hfr_ref, hbr_ref, w_ref, u_ref, b_ref,
             wdf_ref, wdb_ref, bd_ref, out_ref, Hs, Cs, Acc):
        s = pl.program_id(0)

        @pl.when(s == 0)
        def _():
            Hs[...] = jnp.zeros_like(Hs)
            Cs[...] = jnp.zeros_like(Cs)
            Acc[...] = jnp.zeros_like(Acc)

        X = jnp.concatenate(
            [hfs_ref[0], hbs_ref[0], hfr_ref[0], hbr_ref[0]],
            axis=1).astype(jnp.float32)
        Z = (jnp.dot(X, w_ref[...], preferred_element_type=jnp.float32)
             + jnp.dot(Hs[...], u_ref[...], preferred_element_type=jnp.float32)
             + b_ref[...])
        Hn, Cn = _gates(Z, Cs[...], U)
        Cs[...] = Cn
        Hs[...] = Hn
        Wds = jnp.concatenate([wdf_ref[0], wdb_ref[0]], axis=0)
        Acc[...] += jnp.dot(Hn.astype(jnp.bfloat16), Wds,
                            preferred_element_type=jnp.float32)

        @pl.when(s == T - 1)
        def _():
            z = Acc[...] + bd_ref[...]
            lane = lax.broadcasted_iota(jnp.int32, z.shape, 1)
            valid = lane < 3
            zm = jnp.where(valid, z, -jnp.inf)
            m = jnp.max(zm, axis=1, keepdims=True)
            e = jnp.where(valid, jnp.exp(zm - m), 0.0)
            out_ref[...] = e / jnp.sum(e, axis=1, keepdims=True)

    return pl.pallas_call(
        body,
        grid=(T,),
        in_specs=[
            pl.BlockSpec((1, B, U), lambda s: (s, 0, 0)),
            pl.BlockSpec((1, B, U), lambda s: (s, 0, 0)),
            pl.BlockSpec((1, B, U), lambda s: (T - 1 - s, 0, 0)),
            pl.BlockSpec((1, B, U), lambda s: (T - 1 - s, 0, 0)),
            pl.BlockSpec(W2.shape, lambda s: (0, 0)),
            pl.BlockSpec(U2.shape, lambda s: (0, 0)),
            pl.BlockSpec((1, 4 * H2), lambda s: (0, 0)),
            pl.BlockSpec((1, U, 128), lambda s: (s, 0, 0)),
            pl.BlockSpec((1, U, 128), lambda s: (T - 1 - s, 0, 0)),
            pl.BlockSpec((1, 128), lambda s: (0, 0)),
        ],
        out_specs=pl.BlockSpec((B, 128), lambda s: (0, 0)),
        out_shape=jax.ShapeDtypeStruct((B, 128), jnp.float32),
        scratch_shapes=[
            pltpu.VMEM((B, H2), jnp.float32),
            pltpu.VMEM((B, H2), jnp.float32),
            pltpu.VMEM((B, 128), jnp.float32),
        ],
        compiler_params=pltpu.CompilerParams(
            dimension_semantics=("arbitrary",)),
    )(h1f, h1b, h1f, h1b, W2, U2, bc2, Wdf, Wdb, bdp)


def kernel(x, emb, W1f, U1f, b1f, W1b, U1b, b1b,
           W2f, U2f, b2f, W2b, U2b, b2b, Wd, bd):
    B, T = x.shape
    V, D = emb.shape
    U = U1f.shape[0]
    TB = T * B
    NBLK = TB // (_NW * 128)

    idxr = x.T.reshape(_NW, NBLK, 128)
    xe = _sc_gather(idxr, emb, TB, D).reshape(T, B, D)

    Wc1, Uc1, bc1 = _pack_weights(W1f, W1b, U1f, U1b, b1f, b1b, U)
    Wc2, Uc2, bc2 = _pack_weights(W2f, W2b, U2f, U2b, b2f, b2b, U)
    Wd3 = Wd.reshape(T, 2 * U, 3).astype(jnp.bfloat16)
    Wdf = jnp.zeros((T, U, 128), jnp.bfloat16).at[:, :, 0:3].set(Wd3[:, 0:U, :])
    Wdb = jnp.zeros((T, U, 128), jnp.bfloat16).at[:, :, 0:3].set(Wd3[:, U:, :])
    bdp = jnp.zeros((1, 128), jnp.float32).at[0, 0:3].set(bd)

    h1f, h1b = _layer1(xe, Wc1, Uc1, bc1, B, T, D, U)
    probs_pad = _layer2_dense(h1f, h1b, Wc2, Uc2, bc2, Wdf, Wdb, bdp, B, T, U)
    return probs_pad[:, 0:3]


# speedup vs baseline: 3.1011x; 1.0349x over previous
import functools

import jax
import jax.numpy as jnp
import numpy as np
from jax import lax
from jax.experimental import pallas as pl
from jax.experimental.pallas import tpu as pltpu
from jax.experimental.pallas import tpu_sc as plsc

_NC = 2
_NS = 16
_NW = _NC * _NS


def _sc_gather(idxr, emb, TB, D):
    NBLK = idxr.shape[1]
    per_w = NBLK * 128

    mesh = plsc.VectorSubcoreMesh(core_axis_name="c", subcore_axis_name="s")

    @functools.partial(
        pl.kernel,
        out_type=jax.ShapeDtypeStruct((TB, D), jnp.float32),
        mesh=mesh,
        scratch_types=[
            pltpu.VMEM((NBLK, 128), jnp.int32),
            pltpu.VMEM((128, D), jnp.float32),
            pltpu.SemaphoreType.DMA,
        ],
    )
    def gather_k(idx_hbm, emb_hbm, out_hbm, idx_v, rows_v, sem):
        wid = lax.axis_index("s") * _NC + lax.axis_index("c")
        base = wid * per_w
        pltpu.sync_copy(idx_hbm.at[wid], idx_v)

        def body(j, carry):
            pltpu.async_copy(emb_hbm.at[idx_v.at[j]], rows_v, sem).wait()
            pltpu.sync_copy(rows_v, out_hbm.at[pl.ds(base + j * 128, 128)])
            return carry

        lax.fori_loop(0, NBLK, body, 0)

    return gather_k(idxr, emb)


def _pack_weights(Wf, Wb, Uf, Ub, bf, bb, U):
    def inter(Mf, Mb):
        R = Mf.shape[0]
        return jnp.concatenate(
            [Mf.reshape(R, 4, U), Mb.reshape(R, 4, U)], axis=2
        ).reshape(R, 8 * U)

    Wcat = jnp.concatenate([
        inter(Wf, jnp.zeros_like(Wf)),
        inter(jnp.zeros_like(Wb), Wb),
    ], axis=0)
    Ucat = jnp.concatenate([
        inter(Uf, jnp.zeros_like(Uf)),
        inter(jnp.zeros_like(Ub), Ub),
    ], axis=0)
    bc = jnp.concatenate(
        [bf.reshape(4, U), bb.reshape(4, U)], axis=1).reshape(1, 8 * U)
    return Wcat, Ucat, bc


def _sigmoid(x):
    return 0.5 + 0.5 * jnp.tanh(0.5 * x)


def _gates(Z, C, U):
    H2 = 2 * U
    I = _sigmoid(Z[:, 0:H2])
    F = _sigmoid(Z[:, H2:2 * H2])
    G = jnp.tanh(Z[:, 2 * H2:3 * H2])
    O = _sigmoid(Z[:, 3 * H2:4 * H2])
    Cn = F * C + I * G
    Hn = O * jnp.tanh(Cn)
    return Hn, Cn


def _layer1(xe, W1, U1, bc1, B, T, D, U):
    H2 = 2 * U

    def body(xf_ref, xb_ref, w_ref, u_ref, b_ref, hf_ref, hb_ref, Hs, Cs):
        s = pl.program_id(0)

        @pl.when(s == 0)
        def _():
            Hs[...] = jnp.zeros_like(Hs)
            Cs[...] = jnp.zeros_like(Cs)

        X = jnp.concatenate([xf_ref[0], xb_ref[0]], axis=1)
        Z = (jnp.dot(X, w_ref[...], preferred_element_type=jnp.float32)
             + jnp.dot(Hs[...], u_ref[...], preferred_element_type=jnp.float32)
             + b_ref[...])
        Hn, Cn = _gates(Z, Cs[...], U)
        Cs[...] = Cn
        Hs[...] = Hn
        Hb = Hn.astype(jnp.bfloat16)
        hf_ref[0] = Hb[:, 0:U]
        hb_ref[0] = Hb[:, U:H2]

    return pl.pallas_call(
        body,
        grid=(T,),
        in_specs=[
            pl.BlockSpec((1, B, D), lambda s: (s, 0, 0)),
            pl.BlockSpec((1, B, D), lambda s: (T - 1 - s, 0, 0)),
            pl.BlockSpec(W1.shape, lambda s: (0, 0)),
            pl.BlockSpec(U1.shape, lambda s: (0, 0)),
            pl.BlockSpec((1, 4 * H2), lambda s: (0, 0)),
        ],
        out_specs=[
            pl.BlockSpec((1, B, U), lambda s: (s, 0, 0)),
            pl.BlockSpec((1, B, U), lambda s: (T - 1 - s, 0, 0)),
        ],
        out_shape=[
            jax.ShapeDtypeStruct((T, B, U), jnp.bfloat16),
            jax.ShapeDtypeStruct((T, B, U), jnp.bfloat16),
        ],
        scratch_shapes=[
            pltpu.VMEM((B, H2), jnp.float32),
            pltpu.VMEM((B, H2), jnp.float32),
        ],
        compiler_params=pltpu.CompilerParams(
            dimension_semantics=("arbitrary",)),
    )(xe, xe, W1, U1, bc1)


def _layer2_dense(h1f, h1b, W2, U2, bc2, Wdf, Wdb, bdp, B, T, U):
    H2 = 2 * U

    def body(hfs_ref, hbs_ref, hfr_ref, hbr_ref, w_ref, u_ref, b_ref,
             wdf_ref, wdb_ref, bd_ref, out_ref, Hs, Cs, Acc):
        s = pl.program_id(0)

        @pl.when(s == 0)
        def _():
            Hs[...] = jnp.zeros_like(Hs)
            Cs[...] = jnp.zeros_like(Cs)
            Acc[...] = jnp.zeros_like(Acc)

        X = jnp.concatenate(
            [hfs_ref[0], hbs_ref[0], hfr_ref[0], hbr_ref[0]],
            axis=1).astype(jnp.float32)
        Z = (jnp.dot(X, w_ref[...], preferred_element_type=jnp.float32)
             + jnp.dot(Hs[...], u_ref[...], preferred_element_type=jnp.float32)
             + b_ref[...])
        Hn, Cn = _gates(Z, Cs[...], U)
        Cs[...] = Cn
        Hs[...] = Hn
        Wds = jnp.concatenate([wdf_ref[0], wdb_ref[0]], axis=0)
        Acc[...] += jnp.dot(Hn.astype(jnp.bfloat16), Wds,
                            preferred_element_type=jnp.float32)

        @pl.when(s == T - 1)
        def _():
            z = Acc[...] + bd_ref[...]
            lane = lax.broadcasted_iota(jnp.int32, z.shape, 1)
            valid = lane < 3
            zm = jnp.where(valid, z, -jnp.inf)
            m = jnp.max(zm, axis=1, keepdims=True)
            e = jnp.where(valid, jnp.exp(zm - m), 0.0)
            out_ref[...] = e / jnp.sum(e, axis=1, keepdims=True)

    return pl.pallas_call(
        body,
        grid=(T,),
        in_specs=[
            pl.BlockSpec((1, B, U), lambda s: (s, 0, 0)),
            pl.BlockSpec((1, B, U), lambda s: (s, 0, 0)),
            pl.BlockSpec((1, B, U), lambda s: (T - 1 - s, 0, 0)),
            pl.BlockSpec((1, B, U), lambda s: (T - 1 - s, 0, 0)),
            pl.BlockSpec(W2.shape, lambda s: (0, 0)),
            pl.BlockSpec(U2.shape, lambda s: (0, 0)),
            pl.BlockSpec((1, 4 * H2), lambda s: (0, 0)),
            pl.BlockSpec((1, U, 128), lambda s: (s, 0, 0)),
            pl.BlockSpec((1, U, 128), lambda s: (T - 1 - s, 0, 0)),
            pl.BlockSpec((1, 128), lambda s: (0, 0)),
        ],
        out_specs=pl.BlockSpec((B, 128), lambda s: (0, 0)),
        out_shape=jax.ShapeDtypeStruct((B, 128), jnp.float32),
        scratch_shapes=[
            pltpu.VMEM((B, H2), jnp.float32),
            pltpu.VMEM((B, H2), jnp.float32),
            pltpu.VMEM((B, 128), jnp.float32),
        ],
        compiler_params=pltpu.CompilerParams(
            dimension_semantics=("arbitrary",)),
    )(h1f, h1b, h1f, h1b, W2, U2, bc2, Wdf, Wdb, bdp)


def kernel(x, emb, W1f, U1f, b1f, W1b, U1b, b1b,
           W2f, U2f, b2f, W2b, U2b, b2b, Wd, bd):
    B, T = x.shape
    V, D = emb.shape
    U = U1f.shape[0]
    TB = T * B
    NBLK = TB // (_NW * 128)

    idxr = x.T.reshape(_NW, NBLK, 128)
    xe = _sc_gather(idxr, emb, TB, D).reshape(T, B, D)

    Wc1, Uc1, bc1 = _pack_weights(W1f, W1b, U1f, U1b, b1f, b1b, U)
    Wc2, Uc2, bc2 = _pack_weights(W2f, W2b, U2f, U2b, b2f, b2b, U)
    Wd3 = Wd.reshape(T, 2 * U, 3).astype(jnp.bfloat16)
    Wdf = jnp.zeros((T, U, 128), jnp.bfloat16).at[:, :, 0:3].set(Wd3[:, 0:U, :])
    Wdb = jnp.zeros((T, U, 128), jnp.bfloat16).at[:, :, 0:3].set(Wd3[:, U:, :])
    bdp = jnp.zeros((1, 128), jnp.float32).at[0, 0:3].set(bd)

    h1f, h1b = _layer1(xe, Wc1, Uc1, bc1, B, T, D, U)
    probs_pad = _layer2_dense(h1f, h1b, Wc2, Uc2, bc2, Wdf, Wdb, bdp, B, T, U)
    return probs_pad[:, 0:3]


# speedup vs baseline: 3.2818x; 1.0583x over previous
import functools

import jax
import jax.numpy as jnp
import numpy as np
from jax import lax
from jax.experimental import pallas as pl
from jax.experimental.pallas import tpu as pltpu
from jax.experimental.pallas import tpu_sc as plsc

_NC = 2
_NS = 16
_NW = _NC * _NS


def _sc_gather(idxr, emb, TB, D):
    NBLK = idxr.shape[1]
    per_w = NBLK * 128

    mesh = plsc.VectorSubcoreMesh(core_axis_name="c", subcore_axis_name="s")

    @functools.partial(
        pl.kernel,
        out_type=jax.ShapeDtypeStruct((TB, D), jnp.float32),
        mesh=mesh,
        scratch_types=[
            pltpu.VMEM((NBLK, 128), jnp.int32),
            pltpu.VMEM((128, D), jnp.float32),
            pltpu.VMEM((128, D), jnp.float32),
            pltpu.SemaphoreType.DMA,
            pltpu.SemaphoreType.DMA,
        ],
    )
    def gather_k(idx_hbm, emb_hbm, out_hbm, idx_v, rows_a, rows_b, sem_a,
                 sem_b):
        wid = lax.axis_index("s") * _NC + lax.axis_index("c")
        base = wid * per_w
        pltpu.sync_copy(idx_hbm.at[wid], idx_v)

        pltpu.async_copy(emb_hbm.at[idx_v.at[0]], rows_a, sem_a)

        def body(i, carry):
            j0 = 2 * i
            j1 = j0 + 1
            pltpu.async_copy(emb_hbm.at[idx_v.at[j1]], rows_b, sem_b)
            pltpu.make_async_copy(emb_hbm.at[idx_v.at[j0]], rows_a,
                                  sem_a).wait()
            pltpu.sync_copy(rows_a, out_hbm.at[pl.ds(base + j0 * 128, 128)])

            @pl.when(j1 + 1 < NBLK)
            def _():
                pltpu.async_copy(emb_hbm.at[idx_v.at[j1 + 1]], rows_a, sem_a)

            pltpu.make_async_copy(emb_hbm.at[idx_v.at[j1]], rows_b,
                                  sem_b).wait()
            pltpu.sync_copy(rows_b, out_hbm.at[pl.ds(base + j1 * 128, 128)])
            return carry

        lax.fori_loop(0, NBLK // 2, body, 0)

    return gather_k(idxr, emb)


def _pack_weights(Wf, Wb, Uf, Ub, bf, bb, U):
    def inter(Mf, Mb):
        R = Mf.shape[0]
        return jnp.concatenate(
            [Mf.reshape(R, 4, U), Mb.reshape(R, 4, U)], axis=2
        ).reshape(R, 8 * U)

    Wcat = jnp.concatenate([
        inter(Wf, jnp.zeros_like(Wf)),
        inter(jnp.zeros_like(Wb), Wb),
    ], axis=0)
    Ucat = jnp.concatenate([
        inter(Uf, jnp.zeros_like(Uf)),
        inter(jnp.zeros_like(Ub), Ub),
    ], axis=0)
    bc = jnp.concatenate(
        [bf.reshape(4, U), bb.reshape(4, U)], axis=1).reshape(1, 8 * U)
    return Wcat, Ucat, bc


def _sigmoid(x):
    return 0.5 + 0.5 * jnp.tanh(0.5 * x)


def _gates(Z, C, U):
    H2 = 2 * U
    I = _sigmoid(Z[:, 0:H2])
    F = _sigmoid(Z[:, H2:2 * H2])
    G = jnp.tanh(Z[:, 2 * H2:3 * H2])
    O = _sigmoid(Z[:, 3 * H2:4 * H2])
    Cn = F * C + I * G
    Hn = O * jnp.tanh(Cn)
    return Hn, Cn


def _layer1(xe, W1, U1, bc1, B, T, D, U):
    H2 = 2 * U

    def body(xf_ref, xb_ref, w_ref, u_ref, b_ref, hf_ref, hb_ref, Hs, Cs):
        s = pl.program_id(0)

        @pl.when(s == 0)
        def _():
            Hs[...] = jnp.zeros_like(Hs)
            Cs[...] = jnp.zeros_like(Cs)

        X = jnp.concatenate([xf_ref[0], xb_ref[0]], axis=1)
        Z = (jnp.dot(X, w_ref[...], preferred_element_type=jnp.float32)
             + jnp.dot(Hs[...], u_ref[...], preferred_element_type=jnp.float32)
             + b_ref[...])
        Hn, Cn = _gates(Z, Cs[...], U)
        Cs[...] = Cn
        Hs[...] = Hn
        Hb = Hn.astype(jnp.bfloat16)
        hf_ref[0] = Hb[:, 0:U]
        hb_ref[0] = Hb[:, U:H2]

    return pl.pallas_call(
        body,
        grid=(T,),
        in_specs=[
            pl.BlockSpec((1, B, D), lambda s: (s, 0, 0)),
            pl.BlockSpec((1, B, D), lambda s: (T - 1 - s, 0, 0)),
            pl.BlockSpec(W1.shape, lambda s: (0, 0)),
            pl.BlockSpec(U1.shape, lambda s: (0, 0)),
            pl.BlockSpec((1, 4 * H2), lambda s: (0, 0)),
        ],
        out_specs=[
            pl.BlockSpec((1, B, U), lambda s: (s, 0, 0)),
            pl.BlockSpec((1, B, U), lambda s: (T - 1 - s, 0, 0)),
        ],
        out_shape=[
            jax.ShapeDtypeStruct((T, B, U), jnp.bfloat16),
            jax.ShapeDtypeStruct((T, B, U), jnp.bfloat16),
        ],
        scratch_shapes=[
            pltpu.VMEM((B, H2), jnp.float32),
            pltpu.VMEM((B, H2), jnp.float32),
        ],
        compiler_params=pltpu.CompilerParams(
            dimension_semantics=("arbitrary",)),
    )(xe, xe, W1, U1, bc1)


def _layer2_dense(h1f, h1b, W2, U2, bc2, Wdf, Wdb, bdp, B, T, U):
    H2 = 2 * U

    def body(hfs_ref, hbs_ref, hfr_ref, hbr_ref, w_ref, u_ref, b_ref,
             wdf_ref, wdb_ref, bd_ref, out_ref, Hs, Cs, Acc):
        s = pl.program_id(0)

        @pl.when(s == 0)
        def _():
            Hs[...] = jnp.zeros_like(Hs)
            Cs[...] = jnp.zeros_like(Cs)
            Acc[...] = jnp.zeros_like(Acc)

        X = jnp.concatenate(
            [hfs_ref[0], hbs_ref[0], hfr_ref[0], hbr_ref[0]],
            axis=1).astype(jnp.float32)
        Z = (jnp.dot(X, w_ref[...], preferred_element_type=jnp.float32)
             + jnp.dot(Hs[...], u_ref[...], preferred_element_type=jnp.float32)
             + b_ref[...])
        Hn, Cn = _gates(Z, Cs[...], U)
        Cs[...] = Cn
        Hs[...] = Hn
        Wds = jnp.concatenate([wdf_ref[0], wdb_ref[0]], axis=0)
        Acc[...] += jnp.dot(Hn.astype(jnp.bfloat16), Wds,
                            preferred_element_type=jnp.float32)

        @pl.when(s == T - 1)
        def _():
            z = Acc[...] + bd_ref[...]
            lane = lax.broadcasted_iota(jnp.int32, z.shape, 1)
            valid = lane < 3
            zm = jnp.where(valid, z, -jnp.inf)
            m = jnp.max(zm, axis=1, keepdims=True)
            e = jnp.where(valid, jnp.exp(zm - m), 0.0)
            out_ref[...] = e / jnp.sum(e, axis=1, keepdims=True)

    return pl.pallas_call(
        body,
        grid=(T,),
        in_specs=[
            pl.BlockSpec((1, B, U), lambda s: (s, 0, 0)),
            pl.BlockSpec((1, B, U), lambda s: (s, 0, 0)),
            pl.BlockSpec((1, B, U), lambda s: (T - 1 - s, 0, 0)),
            pl.BlockSpec((1, B, U), lambda s: (T - 1 - s, 0, 0)),
            pl.BlockSpec(W2.shape, lambda s: (0, 0)),
            pl.BlockSpec(U2.shape, lambda s: (0, 0)),
            pl.BlockSpec((1, 4 * H2), lambda s: (0, 0)),
            pl.BlockSpec((1, U, 128), lambda s: (s, 0, 0)),
            pl.BlockSpec((1, U, 128), lambda s: (T - 1 - s, 0, 0)),
            pl.BlockSpec((1, 128), lambda s: (0, 0)),
        ],
        out_specs=pl.BlockSpec((B, 128), lambda s: (0, 0)),
        out_shape=jax.ShapeDtypeStruct((B, 128), jnp.float32),
        scratch_shapes=[
            pltpu.VMEM((B, H2), jnp.float32),
            pltpu.VMEM((B, H2), jnp.float32),
            pltpu.VMEM((B, 128), jnp.float32),
        ],
        compiler_params=pltpu.CompilerParams(
            dimension_semantics=("arbitrary",)),
    )(h1f, h1b, h1f, h1b, W2, U2, bc2, Wdf, Wdb, bdp)


def kernel(x, emb, W1f, U1f, b1f, W1b, U1b, b1b,
           W2f, U2f, b2f, W2b, U2b, b2b, Wd, bd):
    B, T = x.shape
    V, D = emb.shape
    U = U1f.shape[0]
    TB = T * B
    NBLK = TB // (_NW * 128)

    idxr = x.T.reshape(_NW, NBLK, 128)
    xe = _sc_gather(idxr, emb, TB, D).reshape(T, B, D)

    Wc1, Uc1, bc1 = _pack_weights(W1f, W1b, U1f, U1b, b1f, b1b, U)
    Wc2, Uc2, bc2 = _pack_weights(W2f, W2b, U2f, U2b, b2f, b2b, U)
    Wd3 = Wd.reshape(T, 2 * U, 3).astype(jnp.bfloat16)
    Wdf = jnp.zeros((T, U, 128), jnp.bfloat16).at[:, :, 0:3].set(Wd3[:, 0:U, :])
    Wdb = jnp.zeros((T, U, 128), jnp.bfloat16).at[:, :, 0:3].set(Wd3[:, U:, :])
    bdp = jnp.zeros((1, 128), jnp.float32).at[0, 0:3].set(bd)

    h1f, h1b = _layer1(xe, Wc1, Uc1, bc1, B, T, D, U)
    probs_pad = _layer2_dense(h1f, h1b, Wc2, Uc2, bc2, Wdf, Wdb, bdp, B, T, U)
    return probs_pad[:, 0:3]
